# baseline (speedup 1.0000x reference)
import jax
import jax.numpy as jnp
from jax.experimental import pallas as pl
from jax.experimental.pallas import tpu as pltpu


def _shift_onehot(w_io):
    w_in = jnp.arange(w_io)
    kw = w_in[:, None] - w_in[None, :] + 3
    return (kw[None, :, :] == jnp.arange(7)[:, None, None]).astype(jnp.float32)


def _toeplitz1(conv_w):
    s = _shift_onehot(32)
    t = jnp.einsum('kab,hkio->hiabo', s, conv_w)
    t = t.reshape(7, 3, 32, 16, 2, 8).transpose(0, 1, 2, 4, 3, 5)
    t = t.reshape(7, 96, 256)
    t = jnp.pad(t, ((0, 0), (0, 32), (0, 0)))
    return t.reshape(7 * 128, 256)


def _toeplitz2(conv_w):
    s = _shift_onehot(16)
    t = jnp.einsum('kab,hkio->haibo', s, conv_w)
    t = t.reshape(7, 16, 8, 8, 2, 16).transpose(0, 1, 2, 4, 3, 5)
    return t.reshape(7 * 128, 256)


def _fused_cnn_body(p_ref, t1_ref, t2_ref, wfc_ref, b1_ref, b2_ref, bfc_ref,
                    out_ref):
    tb = p_ref.shape[2]
    raw = [p_ref[:, t] for t in range(4)]
    z1 = jnp.zeros((1, tb, 128), jnp.bfloat16)

    def piece(o):
        t, c = o % 4, o // 4
        if t == 3:
            if c == 0:
                return raw[0]
            return jnp.concatenate([raw[0][c:8], z1], axis=0)
        src = raw[t + 1]
        if c == 0:
            return jnp.concatenate([z1, src[0:7]], axis=0)
        if c == 1:
            return src[0:8]
        return jnp.concatenate([src[c - 1:8], z1], axis=0)

    pieces_all = [piece(o) for o in range(10)]

    ys = []
    for r in range(4):
        pieces = [pieces_all[r + kh] for kh in range(7)]
        xc = jnp.concatenate(pieces, axis=-1).reshape(8 * tb, 7 * 128)
        ys.append(jnp.dot(xc, t1_ref[...],
                          preferred_element_type=jnp.float32)
                  .reshape(8, tb, 256))
    pe = jnp.maximum(ys[0], ys[1])
    po = jnp.maximum(ys[2], ys[3])
    pe = jnp.maximum(pe[..., :128], pe[..., 128:])
    po = jnp.maximum(po[..., :128], po[..., 128:])
    p1e = jnp.maximum(pe + b1_ref[...], 0.0).astype(jnp.bfloat16)
    p1o = jnp.maximum(po + b1_ref[...], 0.0).astype(jnp.bfloat16)

    z1 = jnp.zeros((1, tb, 128), jnp.bfloat16)
    z2 = jnp.zeros((2, tb, 128), jnp.bfloat16)
    x2 = [jnp.concatenate([z2, p1o, z1], axis=0),
          jnp.concatenate([z1, p1e, z2], axis=0)]
    y2s = []
    for r in range(2):
        pieces = [x2[(r + kh) % 2][(r + kh) // 2:(r + kh) // 2 + 8]
                  for kh in range(7)]
        xc = jnp.concatenate(pieces, axis=-1).reshape(8 * tb, 7 * 128)
        y2s.append(jnp.dot(xc, t2_ref[...],
                           preferred_element_type=jnp.float32)
                   .reshape(8, tb, 256))
    q = jnp.maximum(y2s[0], y2s[1])
    q = jnp.maximum(q[..., :128], q[..., 128:])
    p2 = jnp.maximum(q + b2_ref[...], 0.0).astype(jnp.bfloat16)

    pcat = jnp.concatenate([p2[s] for s in range(8)], axis=-1)
    o = jnp.dot(pcat, wfc_ref[...], preferred_element_type=jnp.float32)
    out_ref[...] = (o + bfc_ref[...]).astype(out_ref.dtype)


def kernel(x_nchw, conv1_w, conv1_b, conv2_w, conv2_b, fc_w, fc_b):
    n = x_nchw.shape[0]
    tb = 256 if n % 256 == 0 else n
    num_classes = fc_w.shape[-1]

    x = x_nchw.astype(jnp.bfloat16).transpose(2, 0, 1, 3).reshape(32, n, 96)
    x = jnp.pad(x, ((0, 0), (0, 0), (0, 32)))
    planes = x.reshape(8, 4, n, 128)

    t1 = _toeplitz1(conv1_w).astype(jnp.bfloat16)
    t2 = _toeplitz2(conv2_w).astype(jnp.bfloat16)
    wfc = (fc_w.reshape(16, 8, 8, num_classes)
           .transpose(1, 2, 0, 3)
           .reshape(1024, num_classes)
           .astype(jnp.bfloat16))
    b1 = jnp.tile(conv1_b, 16).reshape(1, 128)
    b2 = jnp.tile(conv2_b, 8).reshape(1, 128)
    bfc = fc_b.reshape(1, num_classes)

    return pl.pallas_call(
        _fused_cnn_body,
        out_shape=jax.ShapeDtypeStruct((n, num_classes), jnp.float32),
        grid=(n // tb,),
        in_specs=[
            pl.BlockSpec((8, 4, tb, 128), lambda i: (0, 0, i, 0)),
            pl.BlockSpec((7 * 128, 256), lambda i: (0, 0)),
            pl.BlockSpec((7 * 128, 256), lambda i: (0, 0)),
            pl.BlockSpec((1024, num_classes), lambda i: (0, 0)),
            pl.BlockSpec((1, 128), lambda i: (0, 0)),
            pl.BlockSpec((1, 128), lambda i: (0, 0)),
            pl.BlockSpec((1, num_classes), lambda i: (0, 0)),
        ],
        out_specs=pl.BlockSpec((tb, num_classes), lambda i: (i, 0)),
        compiler_params=pltpu.CompilerParams(
            dimension_semantics=("parallel",)),
    )(planes, t1, t2, wfc, b1, b2, bfc)


# speedup vs baseline: 1.0746x; 1.0746x over previous
import jax
import jax.numpy as jnp
from jax.experimental import pallas as pl
from jax.experimental.pallas import tpu as pltpu


def _shift_onehot(w_io):
    w_in = jnp.arange(w_io)
    kw = w_in[:, None] - w_in[None, :] + 3
    return (kw[None, :, :] == jnp.arange(7)[:, None, None]).astype(jnp.float32)


def _toeplitz1(conv_w):
    s = _shift_onehot(32)
    t = jnp.einsum('kab,hkio->hiabo', s, conv_w)
    t = t.reshape(7, 3, 32, 16, 2, 8).transpose(0, 1, 2, 4, 3, 5)
    t = t.reshape(7, 96, 256)
    t = jnp.pad(t, ((0, 0), (0, 32), (0, 0)))
    return t.reshape(7 * 128, 256)


def _toeplitz2(conv_w):
    s = _shift_onehot(16)
    t = jnp.einsum('kab,hkio->haibo', s, conv_w)
    t = t.reshape(7, 16, 8, 8, 2, 16).transpose(0, 1, 2, 4, 3, 5)
    return t.reshape(7 * 128, 256)


def _fused_cnn_body(p_ref, t1_ref, t2_ref, wfc_ref, b1_ref, b2_ref, bfc_ref,
                    out_ref):
    tb = p_ref.shape[2]
    planes = [p_ref[t] for t in range(4)]

    ys = []
    for r in range(4):
        pieces = [planes[(r + kh) % 4][(r + kh) // 4:(r + kh) // 4 + 8]
                  for kh in range(7)]
        xc = jnp.concatenate(pieces, axis=-1).reshape(8 * tb, 7 * 128)
        ys.append(jnp.dot(xc, t1_ref[...],
                          preferred_element_type=jnp.float32)
                  .reshape(8, tb, 256))
    pe = jnp.maximum(ys[0], ys[1])
    po = jnp.maximum(ys[2], ys[3])
    pe = jnp.maximum(pe[..., :128], pe[..., 128:])
    po = jnp.maximum(po[..., :128], po[..., 128:])
    p1e = jnp.maximum(pe + b1_ref[...], 0.0).astype(jnp.bfloat16)
    p1o = jnp.maximum(po + b1_ref[...], 0.0).astype(jnp.bfloat16)

    z1 = jnp.zeros((1, tb, 128), jnp.bfloat16)
    z2 = jnp.zeros((2, tb, 128), jnp.bfloat16)
    x2 = [jnp.concatenate([z2, p1o, z1], axis=0),
          jnp.concatenate([z1, p1e, z2], axis=0)]
    y2s = []
    for r in range(2):
        pieces = [x2[(r + kh) % 2][(r + kh) // 2:(r + kh) // 2 + 8]
                  for kh in range(7)]
        xc = jnp.concatenate(pieces, axis=-1).reshape(8 * tb, 7 * 128)
        y2s.append(jnp.dot(xc, t2_ref[...],
                           preferred_element_type=jnp.float32)
                   .reshape(8, tb, 256))
    q = jnp.maximum(y2s[0], y2s[1])
    q = jnp.maximum(q[..., :128], q[..., 128:])
    p2 = jnp.maximum(q + b2_ref[...], 0.0).astype(jnp.bfloat16)

    pcat = jnp.concatenate([p2[s] for s in range(8)], axis=-1)
    o = jnp.dot(pcat, wfc_ref[...], preferred_element_type=jnp.float32)
    out_ref[...] = (o + bfc_ref[...]).astype(out_ref.dtype)


def kernel(x_nchw, conv1_w, conv1_b, conv2_w, conv2_b, fc_w, fc_b):
    n = x_nchw.shape[0]
    tb = 256 if n % 256 == 0 else n
    num_classes = fc_w.shape[-1]

    x = x_nchw.astype(jnp.float32).transpose(0, 2, 1, 3).reshape(n, 32, 96)
    x = jnp.pad(x, ((0, 0), (3, 5), (0, 32))).astype(jnp.bfloat16)
    planes = x.reshape(n, 10, 4, 128).transpose(2, 1, 0, 3)

    t1 = _toeplitz1(conv1_w).astype(jnp.bfloat16)
    t2 = _toeplitz2(conv2_w).astype(jnp.bfloat16)
    wfc = (fc_w.reshape(16, 8, 8, num_classes)
           .transpose(1, 2, 0, 3)
           .reshape(1024, num_classes)
           .astype(jnp.bfloat16))
    b1 = jnp.tile(conv1_b, 16).reshape(1, 128)
    b2 = jnp.tile(conv2_b, 8).reshape(1, 128)
    bfc = fc_b.reshape(1, num_classes)

    return pl.pallas_call(
        _fused_cnn_body,
        out_shape=jax.ShapeDtypeStruct((n, num_classes), jnp.float32),
        grid=(n // tb,),
        in_specs=[
            pl.BlockSpec((4, 10, tb, 128), lambda i: (0, 0, i, 0)),
            pl.BlockSpec((7 * 128, 256), lambda i: (0, 0)),
            pl.BlockSpec((7 * 128, 256), lambda i: (0, 0)),
            pl.BlockSpec((1024, num_classes), lambda i: (0, 0)),
            pl.BlockSpec((1, 128), lambda i: (0, 0)),
            pl.BlockSpec((1, 128), lambda i: (0, 0)),
            pl.BlockSpec((1, num_classes), lambda i: (0, 0)),
        ],
        out_specs=pl.BlockSpec((tb, num_classes), lambda i: (i, 0)),
        compiler_params=pltpu.CompilerParams(
            dimension_semantics=("parallel",)),
    )(planes, t1, t2, wfc, b1, b2, bfc)
